# Initial kernel scaffold; baseline (speedup 1.0000x reference)
import jax
import jax.numpy as jnp
from jax.experimental import pallas as pl


def kernel(ids, kernel):
    raise NotImplementedError("write your pallas kernel here")


# speedup vs baseline: 1.2092x; 1.2092x over previous
import functools

import jax
import jax.numpy as jnp
from jax import lax
from jax.experimental import pallas as pl
from jax.experimental.pallas import tpu as pltpu
from jax.experimental.pallas import tpu_sc as plsc

N_IDS = 425984
NC = 2
NS = 16
NW = NC * NS
B_PER_W = N_IDS // NW
CHUNK = 128
N_CHUNKS = B_PER_W // CHUNK

_mesh = plsc.VectorSubcoreMesh(core_axis_name="c", subcore_axis_name="s")


@functools.partial(
    pl.kernel,
    mesh=_mesh,
    out_type=jax.ShapeDtypeStruct((NW * N_CHUNKS, CHUNK), jnp.float32),
    scratch_types=[
        pltpu.VMEM((N_CHUNKS, CHUNK), jnp.int32),
        pltpu.VMEM((N_CHUNKS, CHUNK), jnp.float32),
        pltpu.SemaphoreType.DMA,
    ],
)
def _gather_kernel(ids_hbm, table_hbm, out_hbm, idx_v, rows_v, sem):
    wid = lax.axis_index("s") * NC + lax.axis_index("c")
    base = wid * N_CHUNKS
    pltpu.sync_copy(ids_hbm.at[pl.ds(base, N_CHUNKS)], idx_v)

    def fire(j, carry):
        pltpu.async_copy(table_hbm.at[idx_v.at[j]], rows_v.at[j], sem)
        return carry

    lax.fori_loop(0, N_CHUNKS, fire, 0)
    pltpu.make_async_copy(out_hbm.at[pl.ds(base, N_CHUNKS)], rows_v, sem).wait()
    pltpu.sync_copy(rows_v, out_hbm.at[pl.ds(base, N_CHUNKS)])


def kernel(ids, kernel):
    gathered = _gather_kernel(ids.reshape(NW * N_CHUNKS, CHUNK), kernel)
    return gathered.reshape(N_IDS, 1)
